# baseline (speedup 1.0000x reference)
import functools

import jax
import jax.numpy as jnp
from jax import lax
from jax.experimental import pallas as pl
from jax.experimental.pallas import tpu as pltpu
from jax.experimental.pallas import tpu_sc as plsc

N_CHARS = 1000
EMB = 32
W = EMB // 2
PAD_IDX = 0
B = 4096
L = 200
BF = B * L

NC = 2
NS = 16
NW = NC * NS
PER_W = BF // NW
CHUNK = 640
NCH = PER_W // CHUNK
NBUF = 4
LEAD = 2

_mesh = plsc.VectorSubcoreMesh(core_axis_name="c", subcore_axis_name="s")


@functools.partial(
    pl.kernel,
    out_type=jax.ShapeDtypeStruct((BF, W), jnp.int32),
    mesh=_mesh,
    compiler_params=pltpu.CompilerParams(use_tc_tiling_on_sc=False),
    scratch_types=[
        pltpu.VMEM_SHARED((N_CHARS, W), jnp.int32),
        pltpu.VMEM((NCH, CHUNK), jnp.int32),
        pltpu.VMEM((NBUF, CHUNK, W), jnp.int32),
        pltpu.SemaphoreType.DMA((NBUF,)),
        pltpu.SemaphoreType.DMA((NBUF,)),
    ],
)
def _gather_kernel(table_hbm, idx_hbm, out_hbm, table_sh, idx_v, rows_v, gsem, ssem):
    sid = lax.axis_index("s")
    wid = sid * NC + lax.axis_index("c")
    base = wid * PER_W

    @pl.when(sid == 0)
    def _stage():
        pltpu.sync_copy(table_hbm, table_sh)

    pltpu.sync_copy(idx_hbm.at[wid], idx_v)
    plsc.subcore_barrier()

    def start_gather(j, b):
        pltpu.async_copy(table_sh.at[idx_v.at[j]], rows_v.at[b], gsem.at[b])

    def wait_gather(b):
        pltpu.make_async_copy(
            table_sh.at[idx_v.at[0]], rows_v.at[b], gsem.at[b]
        ).wait()

    def start_store(j, b):
        pltpu.async_copy(
            rows_v.at[b], out_hbm.at[pl.ds(base + j * CHUNK, CHUNK)], ssem.at[b]
        )

    def wait_store(b):
        pltpu.make_async_copy(
            rows_v.at[b], out_hbm.at[pl.ds(base, CHUNK)], ssem.at[b]
        ).wait()

    for j in range(LEAD):
        start_gather(j, j)
    for j in range(NBUF - LEAD):
        start_gather(j + LEAD, j + LEAD)
        wait_gather(j)
        start_store(j, j)

    def group(g, carry):
        j0 = (NBUF - LEAD) + g * NBUF
        for b2 in range(NBUF):
            jpar = (NBUF - LEAD) + b2
            b = (jpar + LEAD) % NBUF
            j = j0 + b2
            wait_store(b)
            start_gather(j + LEAD, b)
            wait_gather(jpar % NBUF)
            start_store(j, jpar % NBUF)
        return carry

    lax.fori_loop(0, (NCH - NBUF) // NBUF, group, 0)

    for j in range(NCH - LEAD, NCH):
        wait_gather(j % NBUF)
        start_store(j, j % NBUF)
    for b in range(NBUF):
        wait_store(b)


def kernel(sentence, lengths, table):
    del lengths
    tbl = table.at[PAD_IDX].set(0.0)
    tbl_i = tbl.reshape(N_CHARS, 2, W).transpose(0, 2, 1)
    tbl_w = lax.bitcast_convert_type(
        tbl_i.astype(jnp.bfloat16), jnp.int32
    )
    idx = sentence.reshape(NW, NCH, CHUNK)
    out_w = _gather_kernel(tbl_w, idx)
    lo = lax.bitcast_convert_type(out_w << 16, jnp.float32)
    hi = lax.bitcast_convert_type(out_w & jnp.int32(-65536), jnp.float32)
    return jnp.concatenate([lo, hi], axis=-1).reshape(B, L, EMB)


# speedup vs baseline: 3.2867x; 3.2867x over previous
import functools

import jax
import jax.numpy as jnp
from jax import lax
from jax.experimental import pallas as pl
from jax.experimental.pallas import tpu as pltpu
from jax.experimental.pallas import tpu_sc as plsc

N_CHARS = 1000
EMB = 32
PAD_IDX = 0
B = 4096
L = 200
BF = B * L

NC = 2
NS = 16
NW = NC * NS
PER_W = BF // NW
CHUNK = 640
NCH = PER_W // CHUNK
NBUF = 4
LEAD = 2

_mesh = plsc.VectorSubcoreMesh(core_axis_name="c", subcore_axis_name="s")


@functools.partial(
    pl.kernel,
    out_type=jax.ShapeDtypeStruct((BF, EMB), jnp.float32),
    mesh=_mesh,
    compiler_params=pltpu.CompilerParams(use_tc_tiling_on_sc=False),
    scratch_types=[
        pltpu.VMEM_SHARED((N_CHARS, EMB), jnp.float32),
        pltpu.VMEM((NCH, CHUNK), jnp.int32),
        pltpu.VMEM((NBUF, CHUNK, EMB), jnp.float32),
        pltpu.SemaphoreType.DMA((NBUF,)),
        pltpu.SemaphoreType.DMA((NBUF,)),
    ],
)
def _gather_kernel(table_hbm, idx_hbm, out_hbm, table_sh, idx_v, rows_v, gsem, ssem):
    sid = lax.axis_index("s")
    wid = sid * NC + lax.axis_index("c")
    base = wid * PER_W

    @pl.when(sid == 0)
    def _stage():
        pltpu.sync_copy(table_hbm, table_sh)

    pltpu.sync_copy(idx_hbm.at[wid], idx_v)
    plsc.subcore_barrier()

    def start_gather(j, b):
        pltpu.async_copy(table_sh.at[idx_v.at[j]], rows_v.at[b], gsem.at[b])

    def wait_gather(b):
        pltpu.make_async_copy(
            table_sh.at[idx_v.at[0]], rows_v.at[b], gsem.at[b]
        ).wait()

    def start_store(j, b):
        pltpu.async_copy(
            rows_v.at[b], out_hbm.at[pl.ds(base + j * CHUNK, CHUNK)], ssem.at[b]
        )

    def wait_store(b):
        pltpu.make_async_copy(
            rows_v.at[b], out_hbm.at[pl.ds(base, CHUNK)], ssem.at[b]
        ).wait()

    for j in range(LEAD):
        start_gather(j, j)
    for j in range(NBUF - LEAD):
        start_gather(j + LEAD, j + LEAD)
        wait_gather(j)
        start_store(j, j)

    def group(g, carry):
        j0 = (NBUF - LEAD) + g * NBUF
        for b2 in range(NBUF):
            jpar = (NBUF - LEAD) + b2
            b = (jpar + LEAD) % NBUF
            j = j0 + b2
            wait_store(b)
            start_gather(j + LEAD, b)
            wait_gather(jpar % NBUF)
            start_store(j, jpar % NBUF)
        return carry

    lax.fori_loop(0, (NCH - NBUF) // NBUF, group, 0)

    for j in range(NCH - LEAD, NCH):
        wait_gather(j % NBUF)
        start_store(j, j % NBUF)
    for b in range(NBUF):
        wait_store(b)


def kernel(sentence, lengths, table):
    del lengths
    tbl = table.at[PAD_IDX].set(0.0)
    idx = sentence.reshape(NW, NCH, CHUNK)
    out = _gather_kernel(tbl, idx)
    return out.reshape(B, L, EMB)
